# baseline (speedup 1.0000x reference)
import jax
import jax.numpy as jnp
from jax.experimental import pallas as pl

_ALPHA = 0.2
_TE = 64
_BN = 1000


def _proj_body(h_ref, wt_ref, b_ref, avs_ref, avd_ref, bdr_ref, srep_ref,
               t_ref, adst_ref):
    h = h_ref[...]
    wh = jax.lax.dot_general(h, wt_ref[...], (((1,), (0,)), ((), ())),
                             preferred_element_type=jnp.float32) + b_ref[...]
    srep = srep_ref[...]
    asr = jax.lax.dot_general(wh * avs_ref[...], srep, (((1,), (0,)), ((), ())),
                              preferred_element_type=jnp.float32)
    adr = jax.lax.dot_general(wh * avd_ref[...], srep, (((1,), (0,)), ((), ())),
                              preferred_element_type=jnp.float32) + bdr_ref[...]
    t_ref[:, :256] = wh
    t_ref[:, 256:] = asr
    adst_ref[...] = adr


def _proj(h, wt, b, avs, avd, bdr, srep):
    n = h.shape[0]
    grid = (n // _BN,)
    return pl.pallas_call(
        _proj_body,
        grid=grid,
        in_specs=[
            pl.BlockSpec((_BN, 256), lambda i: (i, 0)),
            pl.BlockSpec((256, 256), lambda i: (0, 0)),
            pl.BlockSpec((1, 256), lambda i: (0, 0)),
            pl.BlockSpec((1, 256), lambda i: (0, 0)),
            pl.BlockSpec((1, 256), lambda i: (0, 0)),
            pl.BlockSpec((1, 256), lambda i: (0, 0)),
            pl.BlockSpec((256, 256), lambda i: (0, 0)),
        ],
        out_specs=[
            pl.BlockSpec((_BN, 512), lambda i: (i, 0)),
            pl.BlockSpec((_BN, 256), lambda i: (i, 0)),
        ],
        out_shape=[
            jax.ShapeDtypeStruct((n, 512), jnp.float32),
            jax.ShapeDtypeStruct((n, 256), jnp.float32),
        ],
    )(h, wt, b, avs, avd, bdr, srep)


def _edge_body(src_ref, dst_ref, t_ref, a2_ref, acc_ref):
    i = pl.program_id(0)
    n = t_ref.shape[0]
    srcv = src_ref[0]
    dstv = dst_ref[0]
    iota = jax.lax.broadcasted_iota(jnp.int32, (n, _TE), 0)
    osrc = (iota == srcv).astype(jnp.float32)
    odst = (iota == dstv).astype(jnp.float32)
    g = jax.lax.dot_general(osrc, t_ref[...], (((0,), (0,)), ((), ())),
                            preferred_element_type=jnp.float32)
    ad = jax.lax.dot_general(odst, a2_ref[...], (((0,), (0,)), ((), ())),
                             preferred_element_type=jnp.float32)
    e = g[:, 256:] + ad
    e = jnp.where(e >= 0, e, _ALPHA * e)
    w = jnp.exp(e)
    vals = g[:, :256] * w
    vv = jnp.concatenate([vals, w], axis=1)
    nc = 1000
    for c in range(n // nc):
        blk = jax.lax.slice(odst, (c * nc, 0), ((c + 1) * nc, _TE))
        upd = jax.lax.dot_general(blk, vv, (((1,), (0,)), ((), ())),
                                  preferred_element_type=jnp.float32)

        @pl.when(i == 0)
        def _(upd=upd, c=c):
            acc_ref[c * nc:(c + 1) * nc, :] = upd

        @pl.when(i > 0)
        def _(upd=upd, c=c):
            acc_ref[c * nc:(c + 1) * nc, :] += upd


def _edge(src3, dst3, t, a2):
    n = t.shape[0]
    nb = src3.shape[0]
    return pl.pallas_call(
        _edge_body,
        grid=(nb,),
        in_specs=[
            pl.BlockSpec((1, 1, _TE), lambda i: (i, 0, 0)),
            pl.BlockSpec((1, 1, _TE), lambda i: (i, 0, 0)),
            pl.BlockSpec((n, 512), lambda i: (0, 0)),
            pl.BlockSpec((n, 256), lambda i: (0, 0)),
        ],
        out_specs=pl.BlockSpec((n, 512), lambda i: (0, 0)),
        out_shape=jax.ShapeDtypeStruct((n, 512), jnp.float32),
    )(src3, dst3, t, a2)


def _norm_body(acc_ref, out_ref):
    a = acc_ref[...]
    out_ref[...] = a[:, :256] / (a[:, 256:] + 1e-16)


def _norm(acc):
    n = acc.shape[0]
    return pl.pallas_call(
        _norm_body,
        grid=(n // _BN,),
        in_specs=[pl.BlockSpec((_BN, 512), lambda i: (i, 0))],
        out_specs=pl.BlockSpec((_BN, 256), lambda i: (i, 0)),
        out_shape=jax.ShapeDtypeStruct((n, 256), jnp.float32),
    )(acc)


def _final_body(x_ref, a2_ref, a3_ref, out_ref):
    a2 = a2_ref[...]
    a3 = a3_ref[...]
    h = (0.25 * (a2[:, :256] / (a2[:, 256:] + 1e-16)) +
         0.25 * (a3[:, :256] / (a3[:, 256:] + 1e-16)))
    out_ref[...] = 0.5 * x_ref[...] + h


def _final(x_out, acc2, acc3):
    n = x_out.shape[0]
    return pl.pallas_call(
        _final_body,
        grid=(n // _BN,),
        in_specs=[
            pl.BlockSpec((_BN, 256), lambda i: (i, 0)),
            pl.BlockSpec((_BN, 512), lambda i: (i, 0)),
            pl.BlockSpec((_BN, 512), lambda i: (i, 0)),
        ],
        out_specs=pl.BlockSpec((_BN, 256), lambda i: (i, 0)),
        out_shape=jax.ShapeDtypeStruct((n, 256), jnp.float32),
    )(x_out, acc2, acc3)


def _prep_weights(W, b, a, ab):
    H, O, F = W.shape
    wt = W.reshape(H * O, F).T
    bf = b.reshape(1, H * O).astype(jnp.float32)
    avs = a[:, :O].reshape(1, H * O).astype(jnp.float32)
    avd = a[:, O:].reshape(1, H * O).astype(jnp.float32)
    bdr = jnp.repeat(ab.astype(jnp.float32), O).reshape(1, H * O)
    return wt.astype(jnp.float32), bf, avs, avd, bdr


def _edges3(ei):
    src = ei[0].astype(jnp.int32).reshape(-1, 1, _TE)
    dst = ei[1].astype(jnp.int32).reshape(-1, 1, _TE)
    return src, dst


def kernel(x, edge_index_intra, hp_counter, edge_index_counter, hp_support,
           edge_index_support, Wg, bg, ag, agb, Wx, bx, ax, axb, t):
    del t
    x = x.astype(jnp.float32)
    idx = jnp.arange(256)
    srep = (idx[:, None] // 32 == idx[None, :] // 32).astype(jnp.float32)

    wtg, bgf, avsg, avdg, bdrg = _prep_weights(Wg, bg, ag, agb)
    wtx, bxf, avsx, avdx, bdrx = _prep_weights(Wx, bx, ax, axb)

    t1, ad1 = _proj(x, wtg, bgf, avsg, avdg, bdrg, srep)
    s1, d1 = _edges3(edge_index_intra)
    acc1 = _edge(s1, d1, t1, ad1)
    x_out = _norm(acc1)

    _, adx = _proj(x_out, wtx, bxf, avsx, avdx, bdrx, srep)
    t2, _ = _proj(hp_counter.astype(jnp.float32), wtx, bxf, avsx, avdx, bdrx, srep)
    t3, _ = _proj(hp_support.astype(jnp.float32), wtx, bxf, avsx, avdx, bdrx, srep)

    s2, d2 = _edges3(edge_index_counter)
    acc2 = _edge(s2, d2, t2, adx)
    s3, d3 = _edges3(edge_index_support)
    acc3 = _edge(s3, d3, t3, adx)

    return _final(x_out, acc2, acc3)


# speedup vs baseline: 2.0480x; 2.0480x over previous
import jax
import jax.numpy as jnp
from jax.experimental import pallas as pl

_ALPHA = 0.2
_TE = 128
_BN = 1000


def _proj_body(h_ref, wt_ref, b_ref, avs_ref, avd_ref, bdr_ref, srep_ref,
               t_ref, adst_ref):
    h = h_ref[...]
    wh = jax.lax.dot_general(h, wt_ref[...], (((1,), (0,)), ((), ())),
                             preferred_element_type=jnp.float32) + b_ref[...]
    srep = srep_ref[...]
    asr = jax.lax.dot_general(wh * avs_ref[...], srep, (((1,), (0,)), ((), ())),
                              preferred_element_type=jnp.float32)
    adr = jax.lax.dot_general(wh * avd_ref[...], srep, (((1,), (0,)), ((), ())),
                              preferred_element_type=jnp.float32) + bdr_ref[...]
    t_ref[:, :256] = wh
    t_ref[:, 256:] = asr
    adst_ref[...] = adr


def _proj(h, wt, b, avs, avd, bdr, srep):
    n = h.shape[0]
    grid = (n // _BN,)
    return pl.pallas_call(
        _proj_body,
        grid=grid,
        in_specs=[
            pl.BlockSpec((_BN, 256), lambda i: (i, 0)),
            pl.BlockSpec((256, 256), lambda i: (0, 0)),
            pl.BlockSpec((1, 256), lambda i: (0, 0)),
            pl.BlockSpec((1, 256), lambda i: (0, 0)),
            pl.BlockSpec((1, 256), lambda i: (0, 0)),
            pl.BlockSpec((1, 256), lambda i: (0, 0)),
            pl.BlockSpec((256, 256), lambda i: (0, 0)),
        ],
        out_specs=[
            pl.BlockSpec((_BN, 512), lambda i: (i, 0)),
            pl.BlockSpec((_BN, 256), lambda i: (i, 0)),
        ],
        out_shape=[
            jax.ShapeDtypeStruct((n, 512), jnp.float32),
            jax.ShapeDtypeStruct((n, 256), jnp.float32),
        ],
    )(h, wt, b, avs, avd, bdr, srep)


def _edge_body(src_ref, dst_ref, t_ref, a2_ref, acc_ref):
    i = pl.program_id(0)
    n = t_ref.shape[0]
    srcv = src_ref[0]
    dstv = dst_ref[0]
    iota = jax.lax.broadcasted_iota(jnp.int32, (n, _TE), 0)
    osrc = (iota == srcv).astype(jnp.float32)
    odst = (iota == dstv).astype(jnp.float32)
    g = jax.lax.dot_general(osrc, t_ref[...], (((0,), (0,)), ((), ())),
                            preferred_element_type=jnp.float32)
    ad = jax.lax.dot_general(odst, a2_ref[...], (((0,), (0,)), ((), ())),
                             preferred_element_type=jnp.float32)
    e = g[:, 256:] + ad
    e = jnp.where(e >= 0, e, _ALPHA * e)
    w = jnp.exp(e)
    vals = g[:, :256] * w
    vv = jnp.concatenate([vals, w], axis=1)
    nc = 1000
    for c in range(n // nc):
        blk = jax.lax.slice(odst, (c * nc, 0), ((c + 1) * nc, _TE))
        upd = jax.lax.dot_general(blk, vv, (((1,), (0,)), ((), ())),
                                  preferred_element_type=jnp.float32)

        @pl.when(i == 0)
        def _(upd=upd, c=c):
            acc_ref[c * nc:(c + 1) * nc, :] = upd

        @pl.when(i > 0)
        def _(upd=upd, c=c):
            acc_ref[c * nc:(c + 1) * nc, :] += upd


def _edge(src3, dst3, t, a2):
    n = t.shape[0]
    nb = src3.shape[0]
    return pl.pallas_call(
        _edge_body,
        grid=(nb,),
        in_specs=[
            pl.BlockSpec((1, 1, _TE), lambda i: (i, 0, 0)),
            pl.BlockSpec((1, 1, _TE), lambda i: (i, 0, 0)),
            pl.BlockSpec((n, 512), lambda i: (0, 0)),
            pl.BlockSpec((n, 256), lambda i: (0, 0)),
        ],
        out_specs=pl.BlockSpec((n, 512), lambda i: (0, 0)),
        out_shape=jax.ShapeDtypeStruct((n, 512), jnp.float32),
    )(src3, dst3, t, a2)


def _norm_body(acc_ref, out_ref):
    a = acc_ref[...]
    out_ref[...] = a[:, :256] / (a[:, 256:] + 1e-16)


def _norm(acc):
    n = acc.shape[0]
    return pl.pallas_call(
        _norm_body,
        grid=(n // _BN,),
        in_specs=[pl.BlockSpec((_BN, 512), lambda i: (i, 0))],
        out_specs=pl.BlockSpec((_BN, 256), lambda i: (i, 0)),
        out_shape=jax.ShapeDtypeStruct((n, 256), jnp.float32),
    )(acc)


def _final_body(x_ref, a2_ref, a3_ref, out_ref):
    a2 = a2_ref[...]
    a3 = a3_ref[...]
    h = (0.25 * (a2[:, :256] / (a2[:, 256:] + 1e-16)) +
         0.25 * (a3[:, :256] / (a3[:, 256:] + 1e-16)))
    out_ref[...] = 0.5 * x_ref[...] + h


def _final(x_out, acc2, acc3):
    n = x_out.shape[0]
    return pl.pallas_call(
        _final_body,
        grid=(n // _BN,),
        in_specs=[
            pl.BlockSpec((_BN, 256), lambda i: (i, 0)),
            pl.BlockSpec((_BN, 512), lambda i: (i, 0)),
            pl.BlockSpec((_BN, 512), lambda i: (i, 0)),
        ],
        out_specs=pl.BlockSpec((_BN, 256), lambda i: (i, 0)),
        out_shape=jax.ShapeDtypeStruct((n, 256), jnp.float32),
    )(x_out, acc2, acc3)


def _prep_weights(W, b, a, ab):
    H, O, F = W.shape
    wt = W.reshape(H * O, F).T
    bf = b.reshape(1, H * O).astype(jnp.float32)
    avs = a[:, :O].reshape(1, H * O).astype(jnp.float32)
    avd = a[:, O:].reshape(1, H * O).astype(jnp.float32)
    bdr = jnp.repeat(ab.astype(jnp.float32), O).reshape(1, H * O)
    return wt.astype(jnp.float32), bf, avs, avd, bdr


def _edges3(ei):
    src = ei[0].astype(jnp.int32).reshape(-1, 1, _TE)
    dst = ei[1].astype(jnp.int32).reshape(-1, 1, _TE)
    return src, dst


def kernel(x, edge_index_intra, hp_counter, edge_index_counter, hp_support,
           edge_index_support, Wg, bg, ag, agb, Wx, bx, ax, axb, t):
    del t
    x = x.astype(jnp.float32)
    idx = jnp.arange(256)
    srep = (idx[:, None] // 32 == idx[None, :] // 32).astype(jnp.float32)

    wtg, bgf, avsg, avdg, bdrg = _prep_weights(Wg, bg, ag, agb)
    wtx, bxf, avsx, avdx, bdrx = _prep_weights(Wx, bx, ax, axb)

    t1, ad1 = _proj(x, wtg, bgf, avsg, avdg, bdrg, srep)
    s1, d1 = _edges3(edge_index_intra)
    acc1 = _edge(s1, d1, t1, ad1)
    x_out = _norm(acc1)

    _, adx = _proj(x_out, wtx, bxf, avsx, avdx, bdrx, srep)
    t2, _ = _proj(hp_counter.astype(jnp.float32), wtx, bxf, avsx, avdx, bdrx, srep)
    t3, _ = _proj(hp_support.astype(jnp.float32), wtx, bxf, avsx, avdx, bdrx, srep)

    s2, d2 = _edges3(edge_index_counter)
    acc2 = _edge(s2, d2, t2, adx)
    s3, d3 = _edges3(edge_index_support)
    acc3 = _edge(s3, d3, t3, adx)

    return _final(x_out, acc2, acc3)


# speedup vs baseline: 3.0186x; 1.4739x over previous
import jax
import jax.numpy as jnp
from jax.experimental import pallas as pl

_ALPHA = 0.2
_TE = 256
_BN = 1000


def _proj_body(h_ref, wt_ref, b_ref, avs_ref, avd_ref, bdr_ref, srep_ref,
               t_ref, adst_ref):
    h = h_ref[...]
    wh = jax.lax.dot_general(h, wt_ref[...], (((1,), (0,)), ((), ())),
                             preferred_element_type=jnp.float32) + b_ref[...]
    srep = srep_ref[...]
    asr = jax.lax.dot_general(wh * avs_ref[...], srep, (((1,), (0,)), ((), ())),
                              preferred_element_type=jnp.float32)
    adr = jax.lax.dot_general(wh * avd_ref[...], srep, (((1,), (0,)), ((), ())),
                              preferred_element_type=jnp.float32) + bdr_ref[...]
    t_ref[:, :256] = wh
    t_ref[:, 256:] = asr
    adst_ref[...] = adr


def _proj(h, wt, b, avs, avd, bdr, srep):
    n = h.shape[0]
    grid = (n // _BN,)
    return pl.pallas_call(
        _proj_body,
        grid=grid,
        in_specs=[
            pl.BlockSpec((_BN, 256), lambda i: (i, 0)),
            pl.BlockSpec((256, 256), lambda i: (0, 0)),
            pl.BlockSpec((1, 256), lambda i: (0, 0)),
            pl.BlockSpec((1, 256), lambda i: (0, 0)),
            pl.BlockSpec((1, 256), lambda i: (0, 0)),
            pl.BlockSpec((1, 256), lambda i: (0, 0)),
            pl.BlockSpec((256, 256), lambda i: (0, 0)),
        ],
        out_specs=[
            pl.BlockSpec((_BN, 512), lambda i: (i, 0)),
            pl.BlockSpec((_BN, 256), lambda i: (i, 0)),
        ],
        out_shape=[
            jax.ShapeDtypeStruct((n, 512), jnp.float32),
            jax.ShapeDtypeStruct((n, 256), jnp.float32),
        ],
    )(h, wt, b, avs, avd, bdr, srep)


def _edge_body(src_ref, dst_ref, t_ref, a2_ref, acc_ref):
    i = pl.program_id(0)
    n = t_ref.shape[0]
    srcv = src_ref[0]
    dstv = dst_ref[0]
    nc = 1000
    g = jnp.zeros((_TE, 512), jnp.float32)
    ad = jnp.zeros((_TE, 256), jnp.float32)
    for c in range(n // nc):
        iota = jax.lax.broadcasted_iota(jnp.int32, (nc, _TE), 0) + c * nc
        osrc = (iota == srcv).astype(jnp.float32)
        odst = (iota == dstv).astype(jnp.float32)
        g = g + jax.lax.dot_general(osrc, t_ref[c * nc:(c + 1) * nc, :],
                                    (((0,), (0,)), ((), ())),
                                    preferred_element_type=jnp.float32)
        ad = ad + jax.lax.dot_general(odst, a2_ref[c * nc:(c + 1) * nc, :],
                                      (((0,), (0,)), ((), ())),
                                      preferred_element_type=jnp.float32)
    e = g[:, 256:] + ad
    e = jnp.where(e >= 0, e, _ALPHA * e)
    w = jnp.exp(e)
    vals = g[:, :256] * w
    vv = jnp.concatenate([vals, w], axis=1)
    for c in range(n // nc):
        iota = jax.lax.broadcasted_iota(jnp.int32, (nc, _TE), 0) + c * nc
        odst = (iota == dstv).astype(jnp.float32)
        upd = jax.lax.dot_general(odst, vv, (((1,), (0,)), ((), ())),
                                  preferred_element_type=jnp.float32)

        @pl.when(i == 0)
        def _(upd=upd, c=c):
            acc_ref[c * nc:(c + 1) * nc, :] = upd

        @pl.when(i > 0)
        def _(upd=upd, c=c):
            acc_ref[c * nc:(c + 1) * nc, :] += upd


def _edge(src3, dst3, t, a2):
    n = t.shape[0]
    nb = src3.shape[0]
    return pl.pallas_call(
        _edge_body,
        grid=(nb,),
        in_specs=[
            pl.BlockSpec((1, 1, _TE), lambda i: (i, 0, 0)),
            pl.BlockSpec((1, 1, _TE), lambda i: (i, 0, 0)),
            pl.BlockSpec((n, 512), lambda i: (0, 0)),
            pl.BlockSpec((n, 256), lambda i: (0, 0)),
        ],
        out_specs=pl.BlockSpec((n, 512), lambda i: (0, 0)),
        out_shape=jax.ShapeDtypeStruct((n, 512), jnp.float32),
    )(src3, dst3, t, a2)


def _norm_body(acc_ref, out_ref):
    a = acc_ref[...]
    out_ref[...] = a[:, :256] / (a[:, 256:] + 1e-16)


def _norm(acc):
    n = acc.shape[0]
    return pl.pallas_call(
        _norm_body,
        grid=(n // _BN,),
        in_specs=[pl.BlockSpec((_BN, 512), lambda i: (i, 0))],
        out_specs=pl.BlockSpec((_BN, 256), lambda i: (i, 0)),
        out_shape=jax.ShapeDtypeStruct((n, 256), jnp.float32),
    )(acc)


def _final_body(x_ref, a2_ref, a3_ref, out_ref):
    a2 = a2_ref[...]
    a3 = a3_ref[...]
    h = (0.25 * (a2[:, :256] / (a2[:, 256:] + 1e-16)) +
         0.25 * (a3[:, :256] / (a3[:, 256:] + 1e-16)))
    out_ref[...] = 0.5 * x_ref[...] + h


def _final(x_out, acc2, acc3):
    n = x_out.shape[0]
    return pl.pallas_call(
        _final_body,
        grid=(n // _BN,),
        in_specs=[
            pl.BlockSpec((_BN, 256), lambda i: (i, 0)),
            pl.BlockSpec((_BN, 512), lambda i: (i, 0)),
            pl.BlockSpec((_BN, 512), lambda i: (i, 0)),
        ],
        out_specs=pl.BlockSpec((_BN, 256), lambda i: (i, 0)),
        out_shape=jax.ShapeDtypeStruct((n, 256), jnp.float32),
    )(x_out, acc2, acc3)


def _prep_weights(W, b, a, ab):
    H, O, F = W.shape
    wt = W.reshape(H * O, F).T
    bf = b.reshape(1, H * O).astype(jnp.float32)
    avs = a[:, :O].reshape(1, H * O).astype(jnp.float32)
    avd = a[:, O:].reshape(1, H * O).astype(jnp.float32)
    bdr = jnp.repeat(ab.astype(jnp.float32), O).reshape(1, H * O)
    return wt.astype(jnp.float32), bf, avs, avd, bdr


def _edges3(ei):
    src = ei[0].astype(jnp.int32).reshape(-1, 1, _TE)
    dst = ei[1].astype(jnp.int32).reshape(-1, 1, _TE)
    return src, dst


def kernel(x, edge_index_intra, hp_counter, edge_index_counter, hp_support,
           edge_index_support, Wg, bg, ag, agb, Wx, bx, ax, axb, t):
    del t
    x = x.astype(jnp.float32)
    idx = jnp.arange(256)
    srep = (idx[:, None] // 32 == idx[None, :] // 32).astype(jnp.float32)

    wtg, bgf, avsg, avdg, bdrg = _prep_weights(Wg, bg, ag, agb)
    wtx, bxf, avsx, avdx, bdrx = _prep_weights(Wx, bx, ax, axb)

    t1, ad1 = _proj(x, wtg, bgf, avsg, avdg, bdrg, srep)
    s1, d1 = _edges3(edge_index_intra)
    acc1 = _edge(s1, d1, t1, ad1)
    x_out = _norm(acc1)

    _, adx = _proj(x_out, wtx, bxf, avsx, avdx, bdrx, srep)
    t2, _ = _proj(hp_counter.astype(jnp.float32), wtx, bxf, avsx, avdx, bdrx, srep)
    t3, _ = _proj(hp_support.astype(jnp.float32), wtx, bxf, avsx, avdx, bdrx, srep)

    s2, d2 = _edges3(edge_index_counter)
    acc2 = _edge(s2, d2, t2, adx)
    s3, d3 = _edges3(edge_index_support)
    acc3 = _edge(s3, d3, t3, adx)

    return _final(x_out, acc2, acc3)
